# baseline (speedup 1.0000x reference)
import functools

import jax
import jax.numpy as jnp
from jax import lax
from jax.experimental import pallas as pl
from jax.experimental.pallas import tpu as pltpu
from jax.experimental.pallas import tpu_sc as plsc

N = 10000
D = 128
E = 320000
NC = 2
NS = 16
NW = NC * NS
EPT = E // NW
CHUNK = 125
STEPS = EPT // CHUNK
NBUF = 2
PH = 4
PSTEPS = STEPS // PH
ROWS_BIG = 640
ROWS_LAST = N - 15 * ROWS_BIG
BM = 2000
GRID_M = N // BM

_mesh = plsc.VectorSubcoreMesh(core_axis_name="c", subcore_axis_name="s")

DEG_CHUNK = 100
DEG_STEPS = EPT // DEG_CHUNK


@functools.partial(
    pl.kernel,
    out_type=jax.ShapeDtypeStruct((2 * N,), jnp.float32),
    mesh=_mesh,
    scratch_types=[
        pltpu.VMEM((DEG_STEPS, DEG_CHUNK), jnp.int32),
        pltpu.VMEM((128,), jnp.float32),
        pltpu.VMEM((ROWS_BIG,), jnp.float32),
        pltpu.VMEM_SHARED((N,), jnp.float32),
        pltpu.SemaphoreType.DMA,
    ],
)
def _deg_kernel(col_hbm, deg_hbm, cidx, ones_v, zbuf, acc, sem):
    cid = lax.axis_index("c")
    sid = lax.axis_index("s")
    wid = cid * NS + sid

    @pl.loop(0, 8)
    def _(j):
        ones_v[pl.ds(j * 16, 16)] = jnp.full((16,), 1.0, jnp.float32)

    @pl.loop(0, ROWS_BIG // 16)
    def _(j):
        zbuf[pl.ds(j * 16, 16)] = jnp.zeros((16,), jnp.float32)

    pltpu.sync_copy(col_hbm.at[wid], cidx)

    @pl.when(sid < 15)
    def _():
        pltpu.sync_copy(zbuf, acc.at[pl.ds(sid * ROWS_BIG, ROWS_BIG)])

    @pl.when(sid == 15)
    def _():
        pltpu.sync_copy(zbuf.at[pl.ds(0, ROWS_LAST)],
                        acc.at[pl.ds(15 * ROWS_BIG, ROWS_LAST)])

    plsc.subcore_barrier()

    @pl.loop(0, DEG_STEPS // 10)
    def _(k):
        for b in range(10):
            pltpu.async_copy(ones_v.at[pl.ds(0, DEG_CHUNK)],
                             acc.at[cidx.at[k * 10 + b]], sem, add=True)
        for b in range(10):
            pltpu.make_async_copy(ones_v.at[pl.ds(0, DEG_CHUNK)],
                                  acc.at[cidx.at[k * 10 + b]], sem).wait()

    plsc.subcore_barrier()

    @pl.when(sid < 15)
    def _():
        pltpu.sync_copy(acc.at[pl.ds(sid * ROWS_BIG, ROWS_BIG)], zbuf)
        pltpu.sync_copy(zbuf,
                        deg_hbm.at[pl.ds(cid * N + sid * ROWS_BIG, ROWS_BIG)])

    @pl.when(sid == 15)
    def _():
        pltpu.sync_copy(acc.at[pl.ds(15 * ROWS_BIG, ROWS_LAST)],
                        zbuf.at[pl.ds(0, ROWS_LAST)])
        pltpu.sync_copy(zbuf.at[pl.ds(0, ROWS_LAST)],
                        deg_hbm.at[pl.ds(cid * N + 15 * ROWS_BIG, ROWS_LAST)])


@functools.partial(
    pl.kernel,
    out_type=jax.ShapeDtypeStruct((2 * N, D), jnp.float32),
    mesh=_mesh,
    scratch_types=[
        pltpu.VMEM((PSTEPS, CHUNK), jnp.int32),
        pltpu.VMEM((PSTEPS, CHUNK), jnp.int32),
        [pltpu.VMEM((CHUNK, D), jnp.float32)] * NBUF,
        pltpu.VMEM_SHARED((N, D), jnp.float32),
        [pltpu.SemaphoreType.DMA] * NBUF,
        [pltpu.SemaphoreType.DMA] * NBUF,
    ],
)
def _agg_kernel(g_hbm, row_hbm, col_hbm, out_hbm, ridx, cidx,
                bufs, acc, gsems, ssems):
    cid = lax.axis_index("c")
    sid = lax.axis_index("s")
    wid = cid * NS + sid
    rstart = sid * ROWS_BIG

    @pl.loop(0, 80)
    def _(i):
        @pl.loop(0, D // 16)
        def _(j):
            bufs[0][i, pl.ds(j * 16, 16)] = jnp.zeros((16,), jnp.float32)

    @pl.when(sid < 15)
    def _():
        @pl.loop(0, ROWS_BIG // 80)
        def _(t):
            pltpu.sync_copy(bufs[0].at[pl.ds(0, 80)],
                            acc.at[pl.ds(rstart + t * 80, 80)])

    @pl.when(sid == 15)
    def _():
        @pl.loop(0, ROWS_LAST // 80)
        def _(t):
            pltpu.sync_copy(bufs[0].at[pl.ds(0, 80)],
                            acc.at[pl.ds(rstart + t * 80, 80)])

    plsc.subcore_barrier()

    def gather(i, b):
        pltpu.async_copy(g_hbm.at[ridx.at[i]], bufs[b], gsems[b])

    def gather_wait(i, b):
        pltpu.make_async_copy(g_hbm.at[ridx.at[i]], bufs[b], gsems[b]).wait()

    def scat(i, b):
        pltpu.async_copy(bufs[b], acc.at[cidx.at[i]], ssems[b], add=True)

    def scat_wait(i, b):
        pltpu.make_async_copy(bufs[b], acc.at[cidx.at[i]], ssems[b]).wait()

    for ph in range(PH):
        pltpu.sync_copy(row_hbm.at[wid, ph], ridx)
        pltpu.sync_copy(col_hbm.at[wid, ph], cidx)

        gather(0, 0)
        gather_wait(0, 0)
        scat(0, 0)
        gather(1, 1)

        @pl.loop(0, (PSTEPS - 2) // NBUF)
        def _(k):
            i0 = k * NBUF + 1
            for u in range(NBUF):
                i = i0 + u
                b = (1 + u) % NBUF
                gather_wait(i, b)
                scat(i, b)
                scat_wait(i - 1, u % NBUF)
                gather(i + 1, u % NBUF)

        i = PSTEPS - 1
        gather_wait(i, i % NBUF)
        scat(i, i % NBUF)
        scat_wait(i - 1, (i - 1) % NBUF)
        scat_wait(i, i % NBUF)

    plsc.subcore_barrier()

    @pl.when(sid < 15)
    def _():
        pltpu.sync_copy(acc.at[pl.ds(rstart, ROWS_BIG)],
                        out_hbm.at[pl.ds(cid * N + rstart, ROWS_BIG)])

    @pl.when(sid == 15)
    def _():
        pltpu.sync_copy(acc.at[pl.ds(rstart, ROWS_LAST)],
                        out_hbm.at[pl.ds(cid * N + rstart, ROWS_LAST)])


def _tc1_body(x_ref, w_ref, da_ref, db_ref, g_ref, dis_ref):
    d = da_ref[...] + db_ref[...]
    dis = jnp.where(d > 0, lax.rsqrt(jnp.maximum(d, 1e-12)), 0.0)
    h = jnp.dot(x_ref[...], w_ref[...], preferred_element_type=jnp.float32)
    g_ref[...] = h * dis
    dis_ref[...] = dis


def _tc1(x, W1, deg_r):
    return pl.pallas_call(
        _tc1_body,
        grid=(GRID_M,),
        in_specs=[
            pl.BlockSpec((BM, D), lambda i: (i, 0)),
            pl.BlockSpec((D, D), lambda i: (0, 0)),
            pl.BlockSpec((BM, 1), lambda i: (i, 0)),
            pl.BlockSpec((BM, 1), lambda i: (i + GRID_M, 0)),
        ],
        out_specs=[
            pl.BlockSpec((BM, D), lambda i: (i, 0)),
            pl.BlockSpec((BM, 1), lambda i: (i, 0)),
        ],
        out_shape=[
            jax.ShapeDtypeStruct((N, D), jnp.float32),
            jax.ShapeDtypeStruct((N, 1), jnp.float32),
        ],
    )(x, W1, deg_r, deg_r)


def _tc2_body(pa_ref, pb_ref, dis_ref, b_ref, w_ref, g_ref):
    dis = dis_ref[...]
    s = (pa_ref[...] + pb_ref[...]) * dis
    t = s + b_ref[...]
    t = jnp.where(t >= 0, t, 0.01 * t)
    h = jnp.dot(t, w_ref[...], preferred_element_type=jnp.float32)
    g_ref[...] = h * dis


def _tc2(p1, dis, b1, W2):
    return pl.pallas_call(
        _tc2_body,
        grid=(GRID_M,),
        in_specs=[
            pl.BlockSpec((BM, D), lambda i: (i, 0)),
            pl.BlockSpec((BM, D), lambda i: (i + GRID_M, 0)),
            pl.BlockSpec((BM, 1), lambda i: (i, 0)),
            pl.BlockSpec((D,), lambda i: (0,)),
            pl.BlockSpec((D, D), lambda i: (0, 0)),
        ],
        out_specs=pl.BlockSpec((BM, D), lambda i: (i, 0)),
        out_shape=jax.ShapeDtypeStruct((N, D), jnp.float32),
    )(p1, p1, dis, b1, W2)


def _tc3_body(pa_ref, pb_ref, dis_ref, b_ref, o_ref):
    s = (pa_ref[...] + pb_ref[...]) * dis_ref[...]
    o_ref[...] = s + b_ref[...]


def _tc3(p2, dis, b2):
    return pl.pallas_call(
        _tc3_body,
        grid=(GRID_M,),
        in_specs=[
            pl.BlockSpec((BM, D), lambda i: (i, 0)),
            pl.BlockSpec((BM, D), lambda i: (i + GRID_M, 0)),
            pl.BlockSpec((BM, 1), lambda i: (i, 0)),
            pl.BlockSpec((D,), lambda i: (0,)),
        ],
        out_specs=pl.BlockSpec((BM, D), lambda i: (i, 0)),
        out_shape=jax.ShapeDtypeStruct((N, D), jnp.float32),
    )(p2, p2, dis, b2)


def kernel(x, edge_index, W1, b1, W2, b2):
    ei = edge_index.astype(jnp.int32)
    row4 = ei[0].reshape(NW, PH, PSTEPS, CHUNK)
    col4 = ei[1].reshape(NW, PH, PSTEPS, CHUNK)
    col3 = ei[1].reshape(NW, DEG_STEPS, DEG_CHUNK)
    deg_flat = _deg_kernel(col3)
    deg_r = deg_flat.reshape(2 * N, 1)
    g1, dis = _tc1(x, W1, deg_r)
    p1 = _agg_kernel(g1, row4, col4)
    g2 = _tc2(p1, dis, b1, W2)
    p2 = _agg_kernel(g2, row4, col4)
    return _tc3(p2, dis, b2)


# speedup vs baseline: 1.1812x; 1.1812x over previous
import functools

import jax
import jax.numpy as jnp
from jax import lax
from jax.experimental import pallas as pl
from jax.experimental.pallas import tpu as pltpu
from jax.experimental.pallas import tpu_sc as plsc

N = 10000
D = 128
E = 320000
NC = 2
NS = 16
NW = NC * NS
EPT = E // NW
CHUNK = 100
STEPS = EPT // CHUNK
NBUF = 3
PH = 4
PSTEPS = STEPS // PH
ROWS_BIG = 640
ROWS_LAST = N - 15 * ROWS_BIG
BM = 2000
GRID_M = N // BM

_mesh = plsc.VectorSubcoreMesh(core_axis_name="c", subcore_axis_name="s")

DEG_CHUNK = 100
DEG_STEPS = EPT // DEG_CHUNK


@functools.partial(
    pl.kernel,
    out_type=jax.ShapeDtypeStruct((2 * N,), jnp.float32),
    mesh=_mesh,
    scratch_types=[
        pltpu.VMEM((DEG_STEPS, DEG_CHUNK), jnp.int32),
        pltpu.VMEM((128,), jnp.float32),
        pltpu.VMEM((ROWS_BIG,), jnp.float32),
        pltpu.VMEM_SHARED((N,), jnp.float32),
        pltpu.SemaphoreType.DMA,
    ],
)
def _deg_kernel(col_hbm, deg_hbm, cidx, ones_v, zbuf, acc, sem):
    cid = lax.axis_index("c")
    sid = lax.axis_index("s")
    wid = cid * NS + sid

    @pl.loop(0, 8)
    def _(j):
        ones_v[pl.ds(j * 16, 16)] = jnp.full((16,), 1.0, jnp.float32)

    @pl.loop(0, ROWS_BIG // 16)
    def _(j):
        zbuf[pl.ds(j * 16, 16)] = jnp.zeros((16,), jnp.float32)

    pltpu.sync_copy(col_hbm.at[wid], cidx)

    @pl.when(sid < 15)
    def _():
        pltpu.sync_copy(zbuf, acc.at[pl.ds(sid * ROWS_BIG, ROWS_BIG)])

    @pl.when(sid == 15)
    def _():
        pltpu.sync_copy(zbuf.at[pl.ds(0, ROWS_LAST)],
                        acc.at[pl.ds(15 * ROWS_BIG, ROWS_LAST)])

    plsc.subcore_barrier()

    @pl.loop(0, DEG_STEPS // 10)
    def _(k):
        for b in range(10):
            pltpu.async_copy(ones_v.at[pl.ds(0, DEG_CHUNK)],
                             acc.at[cidx.at[k * 10 + b]], sem, add=True)
        for b in range(10):
            pltpu.make_async_copy(ones_v.at[pl.ds(0, DEG_CHUNK)],
                                  acc.at[cidx.at[k * 10 + b]], sem).wait()

    plsc.subcore_barrier()

    @pl.when(sid < 15)
    def _():
        pltpu.sync_copy(acc.at[pl.ds(sid * ROWS_BIG, ROWS_BIG)], zbuf)
        pltpu.sync_copy(zbuf,
                        deg_hbm.at[pl.ds(cid * N + sid * ROWS_BIG, ROWS_BIG)])

    @pl.when(sid == 15)
    def _():
        pltpu.sync_copy(acc.at[pl.ds(15 * ROWS_BIG, ROWS_LAST)],
                        zbuf.at[pl.ds(0, ROWS_LAST)])
        pltpu.sync_copy(zbuf.at[pl.ds(0, ROWS_LAST)],
                        deg_hbm.at[pl.ds(cid * N + 15 * ROWS_BIG, ROWS_LAST)])


@functools.partial(
    pl.kernel,
    out_type=jax.ShapeDtypeStruct((2 * N, D), jnp.float32),
    mesh=_mesh,
    scratch_types=[
        pltpu.VMEM((PSTEPS, CHUNK), jnp.int32),
        pltpu.VMEM((PSTEPS, CHUNK), jnp.int32),
        [pltpu.VMEM((CHUNK, D), jnp.float32)] * NBUF,
        pltpu.VMEM_SHARED((N, D), jnp.float32),
        [pltpu.SemaphoreType.DMA] * NBUF,
        [pltpu.SemaphoreType.DMA] * NBUF,
    ],
)
def _agg_kernel(g_hbm, row_hbm, col_hbm, out_hbm, ridx, cidx,
                bufs, acc, gsems, ssems):
    cid = lax.axis_index("c")
    sid = lax.axis_index("s")
    wid = cid * NS + sid
    rstart = sid * ROWS_BIG

    @pl.loop(0, 80)
    def _(i):
        @pl.loop(0, D // 16)
        def _(j):
            bufs[0][i, pl.ds(j * 16, 16)] = jnp.zeros((16,), jnp.float32)

    @pl.when(sid < 15)
    def _():
        @pl.loop(0, ROWS_BIG // 80)
        def _(t):
            pltpu.sync_copy(bufs[0].at[pl.ds(0, 80)],
                            acc.at[pl.ds(rstart + t * 80, 80)])

    @pl.when(sid == 15)
    def _():
        @pl.loop(0, ROWS_LAST // 80)
        def _(t):
            pltpu.sync_copy(bufs[0].at[pl.ds(0, 80)],
                            acc.at[pl.ds(rstart + t * 80, 80)])

    plsc.subcore_barrier()

    def gather(i, b):
        pltpu.async_copy(g_hbm.at[ridx.at[i]], bufs[b], gsems[b])

    def gather_wait(i, b):
        pltpu.make_async_copy(g_hbm.at[ridx.at[i]], bufs[b], gsems[b]).wait()

    def scat(i, b):
        pltpu.async_copy(bufs[b], acc.at[cidx.at[i]], ssems[b], add=True)

    def scat_wait(i, b):
        pltpu.make_async_copy(bufs[b], acc.at[cidx.at[i]], ssems[b]).wait()

    for ph in range(PH):
        pltpu.sync_copy(row_hbm.at[wid, ph], ridx)
        pltpu.sync_copy(col_hbm.at[wid, ph], cidx)

        gather(0, 0)
        gather(1, 1)
        gather_wait(0, 0)
        scat(0, 0)
        gather(2, 2)

        @pl.loop(0, (PSTEPS - NBUF - 1) // NBUF)
        def _(k):
            i0 = k * NBUF + 1
            for u in range(NBUF):
                i = i0 + u
                b = (1 + u) % NBUF
                gather_wait(i, b)
                scat(i, b)
                scat_wait(i - 1, u % NBUF)
                gather(i + 2, u % NBUF)

        i = PSTEPS - 3
        gather_wait(i, i % NBUF)
        scat(i, i % NBUF)
        scat_wait(i - 1, (i - 1) % NBUF)
        gather(i + 2, (i + 2) % NBUF)
        i = PSTEPS - 2
        gather_wait(i, i % NBUF)
        scat(i, i % NBUF)
        scat_wait(i - 1, (i - 1) % NBUF)
        i = PSTEPS - 1
        gather_wait(i, i % NBUF)
        scat(i, i % NBUF)
        scat_wait(i - 1, (i - 1) % NBUF)
        scat_wait(i, i % NBUF)

    plsc.subcore_barrier()

    @pl.when(sid < 15)
    def _():
        pltpu.sync_copy(acc.at[pl.ds(rstart, ROWS_BIG)],
                        out_hbm.at[pl.ds(cid * N + rstart, ROWS_BIG)])

    @pl.when(sid == 15)
    def _():
        pltpu.sync_copy(acc.at[pl.ds(rstart, ROWS_LAST)],
                        out_hbm.at[pl.ds(cid * N + rstart, ROWS_LAST)])


def _tc1_body(x_ref, w_ref, da_ref, db_ref, g_ref, dis_ref):
    d = da_ref[...] + db_ref[...]
    dis = jnp.where(d > 0, lax.rsqrt(jnp.maximum(d, 1e-12)), 0.0)
    h = jnp.dot(x_ref[...], w_ref[...], preferred_element_type=jnp.float32)
    g_ref[...] = h * dis
    dis_ref[...] = dis


def _tc1(x, W1, deg_r):
    return pl.pallas_call(
        _tc1_body,
        grid=(GRID_M,),
        in_specs=[
            pl.BlockSpec((BM, D), lambda i: (i, 0)),
            pl.BlockSpec((D, D), lambda i: (0, 0)),
            pl.BlockSpec((BM, 1), lambda i: (i, 0)),
            pl.BlockSpec((BM, 1), lambda i: (i + GRID_M, 0)),
        ],
        out_specs=[
            pl.BlockSpec((BM, D), lambda i: (i, 0)),
            pl.BlockSpec((BM, 1), lambda i: (i, 0)),
        ],
        out_shape=[
            jax.ShapeDtypeStruct((N, D), jnp.float32),
            jax.ShapeDtypeStruct((N, 1), jnp.float32),
        ],
    )(x, W1, deg_r, deg_r)


def _tc2_body(pa_ref, pb_ref, dis_ref, b_ref, w_ref, g_ref):
    dis = dis_ref[...]
    s = (pa_ref[...] + pb_ref[...]) * dis
    t = s + b_ref[...]
    t = jnp.where(t >= 0, t, 0.01 * t)
    h = jnp.dot(t, w_ref[...], preferred_element_type=jnp.float32)
    g_ref[...] = h * dis


def _tc2(p1, dis, b1, W2):
    return pl.pallas_call(
        _tc2_body,
        grid=(GRID_M,),
        in_specs=[
            pl.BlockSpec((BM, D), lambda i: (i, 0)),
            pl.BlockSpec((BM, D), lambda i: (i + GRID_M, 0)),
            pl.BlockSpec((BM, 1), lambda i: (i, 0)),
            pl.BlockSpec((D,), lambda i: (0,)),
            pl.BlockSpec((D, D), lambda i: (0, 0)),
        ],
        out_specs=pl.BlockSpec((BM, D), lambda i: (i, 0)),
        out_shape=jax.ShapeDtypeStruct((N, D), jnp.float32),
    )(p1, p1, dis, b1, W2)


def _tc3_body(pa_ref, pb_ref, dis_ref, b_ref, o_ref):
    s = (pa_ref[...] + pb_ref[...]) * dis_ref[...]
    o_ref[...] = s + b_ref[...]


def _tc3(p2, dis, b2):
    return pl.pallas_call(
        _tc3_body,
        grid=(GRID_M,),
        in_specs=[
            pl.BlockSpec((BM, D), lambda i: (i, 0)),
            pl.BlockSpec((BM, D), lambda i: (i + GRID_M, 0)),
            pl.BlockSpec((BM, 1), lambda i: (i, 0)),
            pl.BlockSpec((D,), lambda i: (0,)),
        ],
        out_specs=pl.BlockSpec((BM, D), lambda i: (i, 0)),
        out_shape=jax.ShapeDtypeStruct((N, D), jnp.float32),
    )(p2, p2, dis, b2)


def kernel(x, edge_index, W1, b1, W2, b2):
    ei = edge_index.astype(jnp.int32)
    row4 = ei[0].reshape(NW, PH, PSTEPS, CHUNK)
    col4 = ei[1].reshape(NW, PH, PSTEPS, CHUNK)
    col3 = ei[1].reshape(NW, DEG_STEPS, DEG_CHUNK)
    deg_flat = _deg_kernel(col3)
    deg_r = deg_flat.reshape(2 * N, 1)
    g1, dis = _tc1(x, W1, deg_r)
    p1 = _agg_kernel(g1, row4, col4)
    g2 = _tc2(p1, dis, b1, W2)
    p2 = _agg_kernel(g2, row4, col4)
    return _tc3(p2, dis, b2)


# speedup vs baseline: 1.2034x; 1.0188x over previous
import functools

import jax
import jax.numpy as jnp
from jax import lax
from jax.experimental import pallas as pl
from jax.experimental.pallas import tpu as pltpu
from jax.experimental.pallas import tpu_sc as plsc

N = 10000
D = 128
E = 320000
NC = 2
NS = 16
NW = NC * NS
EPT = E // NW
CHUNK = 100
STEPS = EPT // CHUNK
NBUF = 3
PH = 4
PSTEPS = STEPS // PH
ROWS_BIG = 640
ROWS_LAST = N - 15 * ROWS_BIG
BM = 5000
GRID_M = N // BM

_mesh = plsc.VectorSubcoreMesh(core_axis_name="c", subcore_axis_name="s")

DEG_CHUNK = 100
DEG_STEPS = EPT // DEG_CHUNK


@functools.partial(
    pl.kernel,
    out_type=jax.ShapeDtypeStruct((2 * N,), jnp.float32),
    mesh=_mesh,
    scratch_types=[
        pltpu.VMEM((DEG_STEPS, DEG_CHUNK), jnp.int32),
        pltpu.VMEM((128,), jnp.float32),
        pltpu.VMEM((ROWS_BIG,), jnp.float32),
        pltpu.VMEM_SHARED((N,), jnp.float32),
        pltpu.SemaphoreType.DMA,
    ],
)
def _deg_kernel(col_hbm, deg_hbm, cidx, ones_v, zbuf, acc, sem):
    cid = lax.axis_index("c")
    sid = lax.axis_index("s")
    wid = cid * NS + sid

    @pl.loop(0, 8)
    def _(j):
        ones_v[pl.ds(j * 16, 16)] = jnp.full((16,), 1.0, jnp.float32)

    @pl.loop(0, ROWS_BIG // 16)
    def _(j):
        zbuf[pl.ds(j * 16, 16)] = jnp.zeros((16,), jnp.float32)

    pltpu.sync_copy(col_hbm.at[wid], cidx)

    @pl.when(sid < 15)
    def _():
        pltpu.sync_copy(zbuf, acc.at[pl.ds(sid * ROWS_BIG, ROWS_BIG)])

    @pl.when(sid == 15)
    def _():
        pltpu.sync_copy(zbuf.at[pl.ds(0, ROWS_LAST)],
                        acc.at[pl.ds(15 * ROWS_BIG, ROWS_LAST)])

    plsc.subcore_barrier()

    @pl.loop(0, DEG_STEPS // 10)
    def _(k):
        for b in range(10):
            pltpu.async_copy(ones_v.at[pl.ds(0, DEG_CHUNK)],
                             acc.at[cidx.at[k * 10 + b]], sem, add=True)
        for b in range(10):
            pltpu.make_async_copy(ones_v.at[pl.ds(0, DEG_CHUNK)],
                                  acc.at[cidx.at[k * 10 + b]], sem).wait()

    plsc.subcore_barrier()

    @pl.when(sid < 15)
    def _():
        pltpu.sync_copy(acc.at[pl.ds(sid * ROWS_BIG, ROWS_BIG)], zbuf)
        pltpu.sync_copy(zbuf,
                        deg_hbm.at[pl.ds(cid * N + sid * ROWS_BIG, ROWS_BIG)])

    @pl.when(sid == 15)
    def _():
        pltpu.sync_copy(acc.at[pl.ds(15 * ROWS_BIG, ROWS_LAST)],
                        zbuf.at[pl.ds(0, ROWS_LAST)])
        pltpu.sync_copy(zbuf.at[pl.ds(0, ROWS_LAST)],
                        deg_hbm.at[pl.ds(cid * N + 15 * ROWS_BIG, ROWS_LAST)])


@functools.partial(
    pl.kernel,
    out_type=jax.ShapeDtypeStruct((2 * N, D), jnp.float32),
    mesh=_mesh,
    scratch_types=[
        pltpu.VMEM((PSTEPS, CHUNK), jnp.int32),
        pltpu.VMEM((PSTEPS, CHUNK), jnp.int32),
        [pltpu.VMEM((CHUNK, D), jnp.float32)] * NBUF,
        pltpu.VMEM_SHARED((N, D), jnp.float32),
        [pltpu.SemaphoreType.DMA] * NBUF,
        [pltpu.SemaphoreType.DMA] * NBUF,
    ],
)
def _agg_kernel(g_hbm, row_hbm, col_hbm, out_hbm, ridx, cidx,
                bufs, acc, gsems, ssems):
    cid = lax.axis_index("c")
    sid = lax.axis_index("s")
    wid = cid * NS + sid
    rstart = sid * ROWS_BIG

    @pl.loop(0, 80)
    def _(i):
        @pl.loop(0, D // 16)
        def _(j):
            bufs[0][i, pl.ds(j * 16, 16)] = jnp.zeros((16,), jnp.float32)

    @pl.when(sid < 15)
    def _():
        @pl.loop(0, ROWS_BIG // 80)
        def _(t):
            pltpu.sync_copy(bufs[0].at[pl.ds(0, 80)],
                            acc.at[pl.ds(rstart + t * 80, 80)])

    @pl.when(sid == 15)
    def _():
        @pl.loop(0, ROWS_LAST // 80)
        def _(t):
            pltpu.sync_copy(bufs[0].at[pl.ds(0, 80)],
                            acc.at[pl.ds(rstart + t * 80, 80)])

    plsc.subcore_barrier()

    def gather(i, b):
        pltpu.async_copy(g_hbm.at[ridx.at[i]], bufs[b], gsems[b])

    def gather_wait(i, b):
        pltpu.make_async_copy(g_hbm.at[ridx.at[i]], bufs[b], gsems[b]).wait()

    def scat(i, b):
        pltpu.async_copy(bufs[b], acc.at[cidx.at[i]], ssems[b], add=True)

    def scat_wait(i, b):
        pltpu.make_async_copy(bufs[b], acc.at[cidx.at[i]], ssems[b]).wait()

    for ph in range(PH):
        pltpu.sync_copy(row_hbm.at[wid, ph], ridx)
        pltpu.sync_copy(col_hbm.at[wid, ph], cidx)

        gather(0, 0)
        gather(1, 1)
        gather_wait(0, 0)
        scat(0, 0)
        gather(2, 2)

        @pl.loop(0, (PSTEPS - NBUF - 1) // NBUF)
        def _(k):
            i0 = k * NBUF + 1
            for u in range(NBUF):
                i = i0 + u
                b = (1 + u) % NBUF
                gather_wait(i, b)
                scat(i, b)
                scat_wait(i - 1, u % NBUF)
                gather(i + 2, u % NBUF)

        i = PSTEPS - 3
        gather_wait(i, i % NBUF)
        scat(i, i % NBUF)
        scat_wait(i - 1, (i - 1) % NBUF)
        gather(i + 2, (i + 2) % NBUF)
        i = PSTEPS - 2
        gather_wait(i, i % NBUF)
        scat(i, i % NBUF)
        scat_wait(i - 1, (i - 1) % NBUF)
        i = PSTEPS - 1
        gather_wait(i, i % NBUF)
        scat(i, i % NBUF)
        scat_wait(i - 1, (i - 1) % NBUF)
        scat_wait(i, i % NBUF)

    plsc.subcore_barrier()

    @pl.when(sid < 15)
    def _():
        pltpu.sync_copy(acc.at[pl.ds(rstart, ROWS_BIG)],
                        out_hbm.at[pl.ds(cid * N + rstart, ROWS_BIG)])

    @pl.when(sid == 15)
    def _():
        pltpu.sync_copy(acc.at[pl.ds(rstart, ROWS_LAST)],
                        out_hbm.at[pl.ds(cid * N + rstart, ROWS_LAST)])


def _tc1_body(x_ref, w_ref, da_ref, db_ref, g_ref, dis_ref):
    d = da_ref[...] + db_ref[...]
    dis = jnp.where(d > 0, lax.rsqrt(jnp.maximum(d, 1e-12)), 0.0)
    h = jnp.dot(x_ref[...].astype(jnp.bfloat16),
                w_ref[...].astype(jnp.bfloat16),
                preferred_element_type=jnp.float32)
    g_ref[...] = h * dis
    dis_ref[...] = dis


def _tc1(x, W1, deg_r):
    return pl.pallas_call(
        _tc1_body,
        grid=(GRID_M,),
        in_specs=[
            pl.BlockSpec((BM, D), lambda i: (i, 0)),
            pl.BlockSpec((D, D), lambda i: (0, 0)),
            pl.BlockSpec((BM, 1), lambda i: (i, 0)),
            pl.BlockSpec((BM, 1), lambda i: (i + GRID_M, 0)),
        ],
        out_specs=[
            pl.BlockSpec((BM, D), lambda i: (i, 0)),
            pl.BlockSpec((BM, 1), lambda i: (i, 0)),
        ],
        out_shape=[
            jax.ShapeDtypeStruct((N, D), jnp.float32),
            jax.ShapeDtypeStruct((N, 1), jnp.float32),
        ],
    )(x, W1, deg_r, deg_r)


def _tc2_body(pa_ref, pb_ref, dis_ref, b_ref, w_ref, g_ref):
    dis = dis_ref[...]
    s = (pa_ref[...] + pb_ref[...]) * dis
    t = s + b_ref[...]
    t = jnp.where(t >= 0, t, 0.01 * t)
    h = jnp.dot(t.astype(jnp.bfloat16), w_ref[...].astype(jnp.bfloat16),
                preferred_element_type=jnp.float32)
    g_ref[...] = h * dis


def _tc2(p1, dis, b1, W2):
    return pl.pallas_call(
        _tc2_body,
        grid=(GRID_M,),
        in_specs=[
            pl.BlockSpec((BM, D), lambda i: (i, 0)),
            pl.BlockSpec((BM, D), lambda i: (i + GRID_M, 0)),
            pl.BlockSpec((BM, 1), lambda i: (i, 0)),
            pl.BlockSpec((D,), lambda i: (0,)),
            pl.BlockSpec((D, D), lambda i: (0, 0)),
        ],
        out_specs=pl.BlockSpec((BM, D), lambda i: (i, 0)),
        out_shape=jax.ShapeDtypeStruct((N, D), jnp.float32),
    )(p1, p1, dis, b1, W2)


def _tc3_body(pa_ref, pb_ref, dis_ref, b_ref, o_ref):
    s = (pa_ref[...] + pb_ref[...]) * dis_ref[...]
    o_ref[...] = s + b_ref[...]


def _tc3(p2, dis, b2):
    return pl.pallas_call(
        _tc3_body,
        grid=(GRID_M,),
        in_specs=[
            pl.BlockSpec((BM, D), lambda i: (i, 0)),
            pl.BlockSpec((BM, D), lambda i: (i + GRID_M, 0)),
            pl.BlockSpec((BM, 1), lambda i: (i, 0)),
            pl.BlockSpec((D,), lambda i: (0,)),
        ],
        out_specs=pl.BlockSpec((BM, D), lambda i: (i, 0)),
        out_shape=jax.ShapeDtypeStruct((N, D), jnp.float32),
    )(p2, p2, dis, b2)


def kernel(x, edge_index, W1, b1, W2, b2):
    ei = edge_index.astype(jnp.int32)
    row4 = ei[0].reshape(NW, PH, PSTEPS, CHUNK)
    col4 = ei[1].reshape(NW, PH, PSTEPS, CHUNK)
    col3 = ei[1].reshape(NW, DEG_STEPS, DEG_CHUNK)
    deg_flat = _deg_kernel(col3)
    deg_r = deg_flat.reshape(2 * N, 1)
    g1, dis = _tc1(x, W1, deg_r)
    p1 = _agg_kernel(g1, row4, col4)
    g2 = _tc2(p1, dis, b1, W2)
    p2 = _agg_kernel(g2, row4, col4)
    return _tc3(p2, dis, b2)
